# baseline (speedup 1.0000x reference)
import functools

import jax
import jax.numpy as jnp
from jax import lax
from jax.experimental import pallas as pl
from jax.experimental.pallas import tpu as pltpu
from jax.experimental.pallas import tpu_sc as plsc

ROWS = 12
COLS = 31
LANES = 16
NBUF = 3


@functools.lru_cache(maxsize=None)
def _build_sc_kernel(n_tokens, d, t_chunk):
    info = plsc.get_sparse_core_info()
    nc, ns = info.num_cores, info.num_subcores
    nw = nc * ns
    per_w = n_tokens // nw
    n_chunks = per_w // t_chunk
    n_groups = n_chunks // NBUF
    n_tail = n_chunks - n_groups * NBUF
    n_ivec = per_w // LANES
    mesh = plsc.VectorSubcoreMesh(core_axis_name="c", subcore_axis_name="s")

    scratch = [
        pltpu.VMEM((per_w,), jnp.int32),
        pltpu.VMEM((per_w,), jnp.int32),
        pltpu.VMEM((per_w,), jnp.int32),
    ]
    scratch += [pltpu.VMEM((t_chunk, d), jnp.float32) for _ in range(NBUF)]
    scratch += [pltpu.VMEM((t_chunk, d // 2), jnp.int32) for _ in range(NBUF)]
    scratch += [pltpu.SemaphoreType.DMA for _ in range(3 * NBUF)]

    @functools.partial(
        pl.kernel,
        mesh=mesh,
        out_type=jax.ShapeDtypeStruct((n_tokens, d), jnp.float32),
        scratch_types=scratch,
        compiler_params=pltpu.CompilerParams(needs_layout_passes=False),
    )
    def k(src_hbm, r_hbm, c_hbm, table_hbm, out_hbm, r_v, c_v, idx_v, *bufs):
        srcs = bufs[0:NBUF]
        encs = bufs[NBUF:2 * NBUF]
        sems_s = bufs[2 * NBUF:3 * NBUF]
        sems_g = bufs[3 * NBUF:4 * NBUF]
        sems_o = bufs[4 * NBUF:5 * NBUF]
        wid = lax.axis_index("s") * nc + lax.axis_index("c")
        base = wid * per_w

        pltpu.sync_copy(r_hbm.at[pl.ds(base, per_w)], r_v)
        pltpu.sync_copy(c_hbm.at[pl.ds(base, per_w)], c_v)

        def idx_body(u, carry):
            sl = pl.ds(u * LANES, LANES)
            rv = r_v[sl] - 1
            rv = jnp.where(rv < 0, rv + ROWS, rv)
            cv = c_v[sl] - 1
            cv = jnp.where(cv < 0, cv + COLS, cv)
            idx_v[sl] = rv * COLS + cv
            return carry

        lax.fori_loop(0, n_ivec, idx_body, 0)

        def in_copies(ci, m):
            off = base + ci * t_chunk
            cs = pltpu.make_async_copy(
                src_hbm.at[pl.ds(off, t_chunk)], srcs[m], sems_s[m])
            cg = pltpu.make_async_copy(
                table_hbm.at[idx_v.at[pl.ds(ci * t_chunk, t_chunk)]],
                encs[m], sems_g[m])
            return cs, cg

        def issue_in(ci, m):
            cs, cg = in_copies(ci, m)
            cs.start()
            cg.start()

        def wait_in(ci, m):
            cs, cg = in_copies(ci, m)
            cs.wait()
            cg.wait()

        def out_copy(ci, m):
            return pltpu.make_async_copy(
                srcs[m], out_hbm.at[pl.ds(base + ci * t_chunk, t_chunk)],
                sems_o[m])

        def add_chunk(m):
            def body(t, carry):
                for j in range(d // (2 * LANES)):
                    w16 = encs[m][t, pl.ds(j * LANES, LANES)]
                    e32 = plsc.bitcast(w16, jnp.bfloat16)
                    a, b = plsc.unpack(e32, format=plsc.PackFormat.INTERLEAVED)
                    plsc.addupdate(
                        srcs[m].at[t, pl.ds(j * 2 * LANES, LANES)], a)
                    plsc.addupdate(
                        srcs[m].at[t, pl.ds(j * 2 * LANES + LANES, LANES)], b)
                return carry

            lax.fori_loop(0, t_chunk, body, 0)

        def step(ci, m):
            wait_in(ci, m)
            add_chunk(m)
            out_copy(ci, m).start()
            if isinstance(ci, int):
                if ci >= 1:
                    out_copy(ci - 1, (m - 1) % NBUF).wait()
                if ci + 2 < n_chunks:
                    issue_in(ci + 2, (m + 2) % NBUF)
                return

            @pl.when(ci >= 1)
            def _():
                out_copy(ci - 1, (m - 1) % NBUF).wait()

            @pl.when(ci + 2 < n_chunks)
            def _():
                issue_in(ci + 2, (m + 2) % NBUF)

        issue_in(0, 0)
        issue_in(1, 1)

        def group_body(g, carry):
            for m in range(NBUF):
                step(g * NBUF + m, m)
            return carry

        lax.fori_loop(0, n_groups, group_body, 0)
        for e in range(n_tail):
            step(n_groups * NBUF + e, e)
        last = n_chunks - 1
        out_copy(last, last % NBUF).wait()

    return k


TC_BLOCK = 256


@functools.lru_cache(maxsize=None)
def _build_tc_kernel(n_tokens, d):
    nb = n_tokens // TC_BLOCK
    nrows = ROWS * COLS

    def body(s_ref, r_ref, c_ref, t_ref, o_ref):
        rv = r_ref[0] - 1
        rv = jnp.where(rv < 0, rv + ROWS, rv)
        cv = c_ref[0] - 1
        cv = jnp.where(cv < 0, cv + COLS, cv)
        lin = rv * COLS + cv
        iot = lax.broadcasted_iota(jnp.int32, (nrows, TC_BLOCK), 0)
        oh = (iot == lin).astype(jnp.float32)
        enc = lax.dot_general(oh, t_ref[...],
                              (((0,), (0,)), ((), ())),
                              preferred_element_type=jnp.float32)
        o_ref[...] = s_ref[...] + enc

    return pl.pallas_call(
        body,
        grid=(nb,),
        in_specs=[
            pl.BlockSpec((TC_BLOCK, d), lambda i: (i, 0)),
            pl.BlockSpec((1, 1, TC_BLOCK), lambda i: (i, 0, 0)),
            pl.BlockSpec((1, 1, TC_BLOCK), lambda i: (i, 0, 0)),
            pl.BlockSpec((nrows, d), lambda i: (0, 0)),
        ],
        out_specs=pl.BlockSpec((TC_BLOCK, d), lambda i: (i, 0)),
        out_shape=jax.ShapeDtypeStruct((n_tokens, d), jnp.float32),
    )


N_SC = 16384


def kernel(src, dates, encoding):
    b, s, d = src.shape
    n = b * s
    src2 = src.reshape(n, d)
    r = dates[..., 0].astype(jnp.int32).reshape(n)
    c = dates[..., 1].astype(jnp.int32).reshape(n)
    table = encoding.reshape(-1, d).astype(jnp.bfloat16)
    table = (table.reshape(-1, d // 32, 2, LANES)
             .transpose(0, 1, 3, 2).reshape(-1, d // 2, 2))
    table = jax.lax.bitcast_convert_type(table, jnp.int32)
    n_sc = min(N_SC, n)
    out_sc = _build_sc_kernel(n_sc, d, 16)(
        src2[:n_sc], r[:n_sc], c[:n_sc], table)
    if n_sc == n:
        return out_sc.reshape(b, s, d)
    n_tc = n - n_sc
    table_f32 = encoding.reshape(-1, d)
    out_tc = _build_tc_kernel(n_tc, d)(
        src2[n_sc:],
        r[n_sc:].reshape(-1, 1, TC_BLOCK),
        c[n_sc:].reshape(-1, 1, TC_BLOCK),
        table_f32)
    out = jnp.concatenate([out_sc, out_tc], axis=0)
    return out.reshape(b, s, d)


# speedup vs baseline: 1.7137x; 1.7137x over previous
import functools

import jax
import jax.numpy as jnp
from jax import lax
from jax.experimental import pallas as pl
from jax.experimental.pallas import tpu as pltpu
from jax.experimental.pallas import tpu_sc as plsc

ROWS = 12
COLS = 31
LANES = 16
NBUF = 4


@functools.lru_cache(maxsize=None)
def _build_sc_kernel(n_tokens, d, t_chunk):
    info = plsc.get_sparse_core_info()
    nc, ns = info.num_cores, info.num_subcores
    nw = nc * ns
    per_w = n_tokens // nw
    n_chunks = per_w // t_chunk
    n_groups = n_chunks // NBUF
    n_tail = n_chunks - n_groups * NBUF
    n_ivec = per_w // LANES
    mesh = plsc.VectorSubcoreMesh(core_axis_name="c", subcore_axis_name="s")

    scratch = [
        pltpu.VMEM((per_w,), jnp.int32),
        pltpu.VMEM((per_w,), jnp.int32),
        pltpu.VMEM((per_w,), jnp.int32),
    ]
    scratch += [pltpu.VMEM((t_chunk, d), jnp.float32) for _ in range(NBUF)]
    scratch += [pltpu.VMEM((t_chunk, d // 2), jnp.int32) for _ in range(NBUF)]
    scratch += [pltpu.SemaphoreType.DMA for _ in range(3 * NBUF)]

    @functools.partial(
        pl.kernel,
        mesh=mesh,
        out_type=jax.ShapeDtypeStruct((n_tokens, d), jnp.float32),
        scratch_types=scratch,
        compiler_params=pltpu.CompilerParams(needs_layout_passes=False),
    )
    def k(src_hbm, r_hbm, c_hbm, table_hbm, out_hbm, r_v, c_v, idx_v, *bufs):
        srcs = bufs[0:NBUF]
        encs = bufs[NBUF:2 * NBUF]
        sems_s = bufs[2 * NBUF:3 * NBUF]
        sems_g = bufs[3 * NBUF:4 * NBUF]
        sems_o = bufs[4 * NBUF:5 * NBUF]
        wid = lax.axis_index("s") * nc + lax.axis_index("c")
        base = wid * per_w

        pltpu.sync_copy(r_hbm.at[pl.ds(base, per_w)], r_v)
        pltpu.sync_copy(c_hbm.at[pl.ds(base, per_w)], c_v)

        def idx_body(u, carry):
            sl = pl.ds(u * LANES, LANES)
            rv = r_v[sl] - 1
            rv = jnp.where(rv < 0, rv + ROWS, rv)
            cv = c_v[sl] - 1
            cv = jnp.where(cv < 0, cv + COLS, cv)
            idx_v[sl] = rv * COLS + cv
            return carry

        lax.fori_loop(0, n_ivec, idx_body, 0)

        def in_copies(ci, m):
            off = base + ci * t_chunk
            cs = pltpu.make_async_copy(
                src_hbm.at[pl.ds(off, t_chunk)], srcs[m], sems_s[m])
            cg = pltpu.make_async_copy(
                table_hbm.at[idx_v.at[pl.ds(ci * t_chunk, t_chunk)]],
                encs[m], sems_g[m])
            return cs, cg

        def issue_in(ci, m):
            cs, cg = in_copies(ci, m)
            cs.start()
            cg.start()

        def wait_in(ci, m):
            cs, cg = in_copies(ci, m)
            cs.wait()
            cg.wait()

        def out_copy(ci, m):
            return pltpu.make_async_copy(
                srcs[m], out_hbm.at[pl.ds(base + ci * t_chunk, t_chunk)],
                sems_o[m])

        def add_chunk(m):
            def body(t, carry):
                for j in range(d // (2 * LANES)):
                    w16 = encs[m][t, pl.ds(j * LANES, LANES)]
                    e32 = plsc.bitcast(w16, jnp.bfloat16)
                    a, b = plsc.unpack(e32, format=plsc.PackFormat.INTERLEAVED)
                    plsc.addupdate(
                        srcs[m].at[t, pl.ds(j * 2 * LANES, LANES)], a)
                    plsc.addupdate(
                        srcs[m].at[t, pl.ds(j * 2 * LANES + LANES, LANES)], b)
                return carry

            lax.fori_loop(0, t_chunk, body, 0)

        def step(ci, m):
            wait_in(ci, m)
            add_chunk(m)
            out_copy(ci, m).start()
            if isinstance(ci, int):
                if ci >= 1:
                    out_copy(ci - 1, (m - 1) % NBUF).wait()
                if ci + 2 < n_chunks:
                    issue_in(ci + 2, (m + 2) % NBUF)
                return

            @pl.when(ci >= 1)
            def _():
                out_copy(ci - 1, (m - 1) % NBUF).wait()

            @pl.when(ci + 2 < n_chunks)
            def _():
                issue_in(ci + 2, (m + 2) % NBUF)

        issue_in(0, 0)
        issue_in(1, 1)

        def group_body(g, carry):
            for m in range(NBUF):
                step(g * NBUF + m, m)
            return carry

        lax.fori_loop(0, n_groups, group_body, 0)
        for e in range(n_tail):
            step(n_groups * NBUF + e, e)
        last = n_chunks - 1
        out_copy(last, last % NBUF).wait()

    return k


def kernel(src, dates, encoding):
    b, s, d = src.shape
    n = b * s
    src2 = src.reshape(n, d)
    r = dates[..., 0].astype(jnp.int32).reshape(n)
    c = dates[..., 1].astype(jnp.int32).reshape(n)
    table = encoding.reshape(-1, d).astype(jnp.bfloat16)
    table = (table.reshape(-1, d // 32, 2, LANES)
             .transpose(0, 1, 3, 2).reshape(-1, d // 2, 2))
    table = jax.lax.bitcast_convert_type(table, jnp.int32)
    out = _build_sc_kernel(n, d, 16)(src2, r, c, table)
    return out.reshape(b, s, d)


# speedup vs baseline: 1.7162x; 1.0015x over previous
import functools

import jax
import jax.numpy as jnp
from jax import lax
from jax.experimental import pallas as pl
from jax.experimental.pallas import tpu as pltpu
from jax.experimental.pallas import tpu_sc as plsc

ROWS = 12
COLS = 31
LANES = 16
NBUF = 3


@functools.lru_cache(maxsize=None)
def _build_sc_kernel(n_tokens, d, t_chunk):
    info = plsc.get_sparse_core_info()
    nc, ns = info.num_cores, info.num_subcores
    nw = nc * ns
    per_w = n_tokens // nw
    n_chunks = per_w // t_chunk
    n_groups = n_chunks // NBUF
    n_tail = n_chunks - n_groups * NBUF
    n_ivec = per_w // LANES
    mesh = plsc.VectorSubcoreMesh(core_axis_name="c", subcore_axis_name="s")

    scratch = [
        pltpu.VMEM((per_w,), jnp.int32),
        pltpu.VMEM((per_w,), jnp.int32),
        pltpu.VMEM((per_w,), jnp.int32),
    ]
    scratch += [pltpu.VMEM((t_chunk, d), jnp.float32) for _ in range(NBUF)]
    scratch += [pltpu.VMEM((t_chunk, d // 2), jnp.int32) for _ in range(NBUF)]
    scratch += [pltpu.SemaphoreType.DMA for _ in range(3 * NBUF)]

    @functools.partial(
        pl.kernel,
        mesh=mesh,
        out_type=jax.ShapeDtypeStruct((n_tokens, d), jnp.float32),
        scratch_types=scratch,
        compiler_params=pltpu.CompilerParams(needs_layout_passes=False),
    )
    def k(src_hbm, r_hbm, c_hbm, table_hbm, out_hbm, r_v, c_v, idx_v, *bufs):
        srcs = bufs[0:NBUF]
        encs = bufs[NBUF:2 * NBUF]
        sems_s = bufs[2 * NBUF:3 * NBUF]
        sems_g = bufs[3 * NBUF:4 * NBUF]
        sems_o = bufs[4 * NBUF:5 * NBUF]
        wid = lax.axis_index("s") * nc + lax.axis_index("c")
        base = wid * per_w

        pltpu.sync_copy(r_hbm.at[pl.ds(base, per_w)], r_v)
        pltpu.sync_copy(c_hbm.at[pl.ds(base, per_w)], c_v)

        def idx_body(u, carry):
            sl = pl.ds(u * LANES, LANES)
            rv = r_v[sl] - 1
            rv = jnp.where(rv < 0, rv + ROWS, rv)
            cv = c_v[sl] - 1
            cv = jnp.where(cv < 0, cv + COLS, cv)
            idx_v[sl] = rv * COLS + cv
            return carry

        lax.fori_loop(0, n_ivec, idx_body, 0)

        def in_copies(ci, m):
            off = base + ci * t_chunk
            cs = pltpu.make_async_copy(
                src_hbm.at[pl.ds(off, t_chunk)], srcs[m], sems_s[m])
            cg = pltpu.make_async_copy(
                table_hbm.at[idx_v.at[pl.ds(ci * t_chunk, t_chunk)]],
                encs[m], sems_g[m])
            return cs, cg

        def issue_in(ci, m):
            cs, cg = in_copies(ci, m)
            cs.start()
            cg.start()

        def wait_in(ci, m):
            cs, cg = in_copies(ci, m)
            cs.wait()
            cg.wait()

        def out_copy(ci, m):
            return pltpu.make_async_copy(
                srcs[m], out_hbm.at[pl.ds(base + ci * t_chunk, t_chunk)],
                sems_o[m])

        def add_chunk(m):
            def body(t, carry):
                for j in range(d // (2 * LANES)):
                    w16 = encs[m][t, pl.ds(j * LANES, LANES)]
                    e32 = plsc.bitcast(w16, jnp.bfloat16)
                    a, b = plsc.unpack(e32, format=plsc.PackFormat.INTERLEAVED)
                    plsc.addupdate(
                        srcs[m].at[t, pl.ds(j * 2 * LANES, LANES)], a)
                    plsc.addupdate(
                        srcs[m].at[t, pl.ds(j * 2 * LANES + LANES, LANES)], b)
                return carry

            lax.fori_loop(0, t_chunk, body, 0)

        def step(ci, m):
            wait_in(ci, m)
            add_chunk(m)
            out_copy(ci, m).start()
            if isinstance(ci, int):
                if ci >= 1:
                    out_copy(ci - 1, (m - 1) % NBUF).wait()
                if ci + 2 < n_chunks:
                    issue_in(ci + 2, (m + 2) % NBUF)
                return

            @pl.when(ci >= 1)
            def _():
                out_copy(ci - 1, (m - 1) % NBUF).wait()

            @pl.when(ci + 2 < n_chunks)
            def _():
                issue_in(ci + 2, (m + 2) % NBUF)

        issue_in(0, 0)
        issue_in(1, 1)

        def group_body(g, carry):
            for m in range(NBUF):
                step(g * NBUF + m, m)
            return carry

        lax.fori_loop(0, n_groups, group_body, 0)
        for e in range(n_tail):
            step(n_groups * NBUF + e, e)
        last = n_chunks - 1
        out_copy(last, last % NBUF).wait()

    return k


def kernel(src, dates, encoding):
    b, s, d = src.shape
    n = b * s
    src2 = src.reshape(n, d)
    r = dates[..., 0].astype(jnp.int32).reshape(n)
    c = dates[..., 1].astype(jnp.int32).reshape(n)
    table = encoding.reshape(-1, d).astype(jnp.bfloat16)
    table = (table.reshape(-1, d // 32, 2, LANES)
             .transpose(0, 1, 3, 2).reshape(-1, d // 2, 2))
    table = jax.lax.bitcast_convert_type(table, jnp.int32)
    out = _build_sc_kernel(n, d, 16)(src2, r, c, table)
    return out.reshape(b, s, d)
